# Initial kernel scaffold; baseline (speedup 1.0000x reference)
import jax
import jax.numpy as jnp
from jax.experimental import pallas as pl


def kernel(x, W1, b1, W2, b2, W3, b3, edge_src, edge_dst, edge_kidx):
    raise NotImplementedError("write your pallas kernel here")


# speedup vs baseline: 4.5835x; 4.5835x over previous
import functools

import jax
import jax.numpy as jnp
from jax import lax
from jax.experimental import pallas as pl
from jax.experimental.pallas import tpu as pltpu
from jax.experimental.pallas import tpu_sc as plsc

_N = 10000
_E = 160000
_NIN = 128
_NOUT = 128
_CB = 32
_K = 27

_NC = 2
_NS = 16
_L = 16

_EPAD = 163840
_EPT = _EPAD // (_NC * _NS)
_CHUNK = 128
_NCH = _EPT // _CHUNK
_NACC = 10240
_JUNK = 10200
_RPT = _NACC // _NS
_ZR = 64


def _tc1_body(x_ref, w1_ref, b1_ref, w2_ref, ht_ref):
    h = jnp.dot(x_ref[...], w1_ref[...], preferred_element_type=jnp.float32)
    h = jnp.maximum(h + b1_ref[...], 0.0)
    for k in range(_K):
        ht_ref[k] = jnp.dot(h, w2_ref[k], preferred_element_type=jnp.float32)


def _tc2_body(p_ref, x_ref, b2_ref, w3_ref, b3_ref, o_ref):
    h2 = jnp.maximum(p_ref[0] + p_ref[1] + b2_ref[...], 0.0)
    h3 = jnp.dot(h2, w3_ref[...], preferred_element_type=jnp.float32)
    h3 = h3 + b3_ref[...]
    o_ref[...] = jnp.maximum((h3 + x_ref[...]) * 0.5, 0.0)


def _sc_body(ht_hbm, src_hbm, kidx_hbm, dst_hbm, out_hbm,
             srcbuf, kidxbuf, gidx, dstbuf, rows, zbuf, acc, sem):
    c = lax.axis_index("c")
    s = lax.axis_index("s")
    wid = c * _NS + s
    ebase = wid * _EPT

    pltpu.sync_copy(src_hbm.at[pl.ds(ebase, _EPT)], srcbuf)
    pltpu.sync_copy(kidx_hbm.at[pl.ds(ebase, _EPT)], kidxbuf)
    pltpu.sync_copy(dst_hbm.at[pl.ds(wid * _NCH, _NCH)], dstbuf)

    def build_idx(i, carry):
        o = i * _L
        gidx[pl.ds(o, _L)] = kidxbuf[pl.ds(o, _L)] * _N + srcbuf[pl.ds(o, _L)]
        return carry
    lax.fori_loop(0, _EPT // _L, build_idx, 0)

    z16 = jnp.zeros((_L,), jnp.float32)

    def zero_zbuf(r, carry):
        zbuf[r, pl.ds(0, _L)] = z16
        zbuf[r, pl.ds(_L, _L)] = z16
        return carry
    lax.fori_loop(0, _ZR, zero_zbuf, 0)

    def zero_acc(t, carry):
        pltpu.sync_copy(zbuf, acc.at[pl.ds(s * _RPT + t * _ZR, _ZR)])
        return carry
    lax.fori_loop(0, _RPT // _ZR, zero_acc, 0)
    plsc.subcore_barrier()

    def chunk(j, carry):
        pltpu.async_copy(
            ht_hbm.at[gidx.at[pl.ds(j * _CHUNK, _CHUNK)]], rows, sem).wait()
        pltpu.sync_copy(rows, acc.at[dstbuf.at[j]], add=True)
        return carry
    lax.fori_loop(0, _NCH, chunk, 0)
    plsc.subcore_barrier()

    pltpu.sync_copy(acc.at[pl.ds(s * _RPT, _RPT)],
                    out_hbm.at[pl.ds(c * _NACC + s * _RPT, _RPT)])


@jax.jit
def kernel(x, W1, b1, W2, b2, W3, b3, edge_src, edge_dst, edge_kidx):
    npad = _EPAD - _E
    src_p = jnp.concatenate([edge_src, jnp.zeros((npad,), jnp.int32)])
    kidx_p = jnp.concatenate([edge_kidx, jnp.zeros((npad,), jnp.int32)])
    dst_p = jnp.concatenate(
        [edge_dst, jnp.full((npad,), _JUNK, jnp.int32)]).reshape(-1, _CHUNK)

    bn1 = 512
    grid1 = pl.cdiv(_N, bn1)
    ht = pl.pallas_call(
        _tc1_body,
        grid=(grid1,),
        in_specs=[
            pl.BlockSpec((bn1, _NIN), lambda i: (i, 0)),
            pl.BlockSpec((_NIN, _CB), lambda i: (0, 0)),
            pl.BlockSpec((1, _CB), lambda i: (0, 0)),
            pl.BlockSpec((_K, _CB, _CB), lambda i: (0, 0, 0)),
        ],
        out_specs=pl.BlockSpec((_K, bn1, _CB), lambda i: (0, i, 0)),
        out_shape=jax.ShapeDtypeStruct((_K, _N, _CB), jnp.float32),
    )(x, W1, b1.reshape(1, _CB), W2)
    ht_rows = ht.reshape(_K * _N, _CB)

    mesh = plsc.VectorSubcoreMesh(core_axis_name="c", subcore_axis_name="s")
    partial = pl.kernel(
        _sc_body,
        out_type=jax.ShapeDtypeStruct((_NC * _NACC, _CB), jnp.float32),
        mesh=mesh,
        compiler_params=pltpu.CompilerParams(use_tc_tiling_on_sc=False),
        scratch_types=[
            pltpu.VMEM((_EPT,), jnp.int32),
            pltpu.VMEM((_EPT,), jnp.int32),
            pltpu.VMEM((_EPT,), jnp.int32),
            pltpu.VMEM((_NCH, _CHUNK), jnp.int32),
            pltpu.VMEM((_CHUNK, _CB), jnp.float32),
            pltpu.VMEM((_ZR, _CB), jnp.float32),
            pltpu.VMEM_SHARED((_NACC, _CB), jnp.float32),
            pltpu.SemaphoreType.DMA,
        ],
    )(ht_rows, src_p, kidx_p, dst_p)
    psum = partial.reshape(_NC, _NACC, _CB)

    bn2 = 512
    grid2 = pl.cdiv(_N, bn2)
    out = pl.pallas_call(
        _tc2_body,
        grid=(grid2,),
        in_specs=[
            pl.BlockSpec((_NC, bn2, _CB), lambda i: (0, i, 0)),
            pl.BlockSpec((bn2, _NIN), lambda i: (i, 0)),
            pl.BlockSpec((1, _CB), lambda i: (0, 0)),
            pl.BlockSpec((_CB, _NOUT), lambda i: (0, 0)),
            pl.BlockSpec((1, _NOUT), lambda i: (0, 0)),
        ],
        out_specs=pl.BlockSpec((bn2, _NOUT), lambda i: (i, 0)),
        out_shape=jax.ShapeDtypeStruct((_N, _NOUT), jnp.float32),
    )(psum, x, b2.reshape(1, _CB), W3, b3.reshape(1, _NOUT))
    return out


# speedup vs baseline: 6.8125x; 1.4863x over previous
import functools

import jax
import jax.numpy as jnp
from jax import lax
from jax.experimental import pallas as pl
from jax.experimental.pallas import tpu as pltpu
from jax.experimental.pallas import tpu_sc as plsc

_N = 10000
_E = 160000
_NIN = 128
_NOUT = 128
_CB = 32
_K = 27

_NC = 2
_NS = 16
_L = 16

_EPAD = 163840
_EPT = _EPAD // (_NC * _NS)
_CHUNK = 128
_NCH = _EPT // _CHUNK
_NACC = 10240
_JUNK = 10200
_RPT = _NACC // _NS
_ZR = 64


def _tc1_body(x_ref, w1_ref, b1_ref, w2_ref, ht_ref):
    h = jnp.dot(x_ref[...], w1_ref[...], preferred_element_type=jnp.float32)
    h = jnp.maximum(h + b1_ref[...], 0.0)
    ht_ref[...] = jnp.dot(h, w2_ref[...], preferred_element_type=jnp.float32)


def _tc2_body(p_ref, x_ref, b2_ref, w3_ref, b3_ref, o_ref):
    h2 = jnp.maximum(p_ref[0] + p_ref[1] + b2_ref[...], 0.0)
    h3 = jnp.dot(h2, w3_ref[...], preferred_element_type=jnp.float32)
    h3 = h3 + b3_ref[...]
    o_ref[...] = jnp.maximum((h3 + x_ref[...]) * 0.5, 0.0)


def _sc_body(ht_hbm, src_hbm, kidx_hbm, dst_hbm, out_hbm,
             srcbuf, kidxbuf, gidx, dstbuf, rows0, rows1, zbuf, acc,
             sem0, sem1):
    c = lax.axis_index("c")
    s = lax.axis_index("s")
    wid = c * _NS + s
    ebase = wid * _EPT

    pltpu.sync_copy(src_hbm.at[pl.ds(ebase, _EPT)], srcbuf)
    pltpu.sync_copy(kidx_hbm.at[pl.ds(ebase, _EPT)], kidxbuf)
    pltpu.sync_copy(dst_hbm.at[pl.ds(wid * _NCH, _NCH)], dstbuf)

    def build_idx(i, carry):
        o = i * _L
        gidx[pl.ds(o, _L)] = srcbuf[pl.ds(o, _L)] * _K + kidxbuf[pl.ds(o, _L)]
        return carry
    lax.fori_loop(0, _EPT // _L, build_idx, 0)

    z16 = jnp.zeros((_L,), jnp.float32)

    def zero_zbuf(r, carry):
        zbuf[r, pl.ds(0, _L)] = z16
        zbuf[r, pl.ds(_L, _L)] = z16
        return carry
    lax.fori_loop(0, _ZR, zero_zbuf, 0)

    def zero_acc(t, carry):
        pltpu.sync_copy(zbuf, acc.at[pl.ds(s * _RPT + t * _ZR, _ZR)])
        return carry
    lax.fori_loop(0, _RPT // _ZR, zero_acc, 0)
    plsc.subcore_barrier()

    def gather(j, rb, sb):
        pltpu.async_copy(ht_hbm.at[gidx.at[pl.ds(j * _CHUNK, _CHUNK)]],
                         rb, sb)

    def wait(rb, sb):
        pltpu.make_async_copy(ht_hbm.at[pl.ds(0, _CHUNK)], rb, sb).wait()

    gather(0, rows0, sem0)

    def chunk_pair(jj, carry):
        j = jj * 2
        gather(j + 1, rows1, sem1)
        wait(rows0, sem0)
        pltpu.sync_copy(rows0, acc.at[dstbuf.at[j]], add=True)

        @pl.when(j + 2 < _NCH)
        def _():
            gather(j + 2, rows0, sem0)
        wait(rows1, sem1)
        pltpu.sync_copy(rows1, acc.at[dstbuf.at[j + 1]], add=True)
        return carry
    lax.fori_loop(0, _NCH // 2, chunk_pair, 0)
    plsc.subcore_barrier()

    pltpu.sync_copy(acc.at[pl.ds(s * _RPT, _RPT)],
                    out_hbm.at[pl.ds(c * _NACC + s * _RPT, _RPT)])


@jax.jit
def kernel(x, W1, b1, W2, b2, W3, b3, edge_src, edge_dst, edge_kidx):
    npad = _EPAD - _E
    src_p = jnp.concatenate([edge_src, jnp.zeros((npad,), jnp.int32)])
    kidx_p = jnp.concatenate([edge_kidx, jnp.zeros((npad,), jnp.int32)])
    dst_p = jnp.concatenate(
        [edge_dst, jnp.full((npad,), _JUNK, jnp.int32)]).reshape(-1, _CHUNK)

    bn1 = 512
    grid1 = pl.cdiv(_N, bn1)
    w2all = jnp.transpose(W2, (1, 0, 2)).reshape(_CB, _K * _CB)
    ht = pl.pallas_call(
        _tc1_body,
        grid=(grid1,),
        in_specs=[
            pl.BlockSpec((bn1, _NIN), lambda i: (i, 0)),
            pl.BlockSpec((_NIN, _CB), lambda i: (0, 0)),
            pl.BlockSpec((1, _CB), lambda i: (0, 0)),
            pl.BlockSpec((_CB, _K * _CB), lambda i: (0, 0)),
        ],
        out_specs=pl.BlockSpec((bn1, _K * _CB), lambda i: (i, 0)),
        out_shape=jax.ShapeDtypeStruct((_N, _K * _CB), jnp.float32),
    )(x, W1, b1.reshape(1, _CB), w2all)
    ht_rows = ht.reshape(_N * _K, _CB)

    mesh = plsc.VectorSubcoreMesh(core_axis_name="c", subcore_axis_name="s")
    partial = pl.kernel(
        _sc_body,
        out_type=jax.ShapeDtypeStruct((_NC * _NACC, _CB), jnp.float32),
        mesh=mesh,
        compiler_params=pltpu.CompilerParams(use_tc_tiling_on_sc=False),
        scratch_types=[
            pltpu.VMEM((_EPT,), jnp.int32),
            pltpu.VMEM((_EPT,), jnp.int32),
            pltpu.VMEM((_EPT,), jnp.int32),
            pltpu.VMEM((_NCH, _CHUNK), jnp.int32),
            pltpu.VMEM((_CHUNK, _CB), jnp.float32),
            pltpu.VMEM((_CHUNK, _CB), jnp.float32),
            pltpu.VMEM((_ZR, _CB), jnp.float32),
            pltpu.VMEM_SHARED((_NACC, _CB), jnp.float32),
            pltpu.SemaphoreType.DMA,
            pltpu.SemaphoreType.DMA,
        ],
    )(ht_rows, src_p, kidx_p, dst_p)
    psum = partial.reshape(_NC, _NACC, _CB)

    bn2 = 512
    grid2 = pl.cdiv(_N, bn2)
    out = pl.pallas_call(
        _tc2_body,
        grid=(grid2,),
        in_specs=[
            pl.BlockSpec((_NC, bn2, _CB), lambda i: (0, i, 0)),
            pl.BlockSpec((bn2, _NIN), lambda i: (i, 0)),
            pl.BlockSpec((1, _CB), lambda i: (0, 0)),
            pl.BlockSpec((_CB, _NOUT), lambda i: (0, 0)),
            pl.BlockSpec((1, _NOUT), lambda i: (0, 0)),
        ],
        out_specs=pl.BlockSpec((bn2, _NOUT), lambda i: (i, 0)),
        out_shape=jax.ShapeDtypeStruct((_N, _NOUT), jnp.float32),
    )(psum, x, b2.reshape(1, _CB), W3, b3.reshape(1, _NOUT))
    return out


# speedup vs baseline: 9.0322x; 1.3258x over previous
import functools

import jax
import jax.numpy as jnp
from jax import lax
from jax.experimental import pallas as pl
from jax.experimental.pallas import tpu as pltpu
from jax.experimental.pallas import tpu_sc as plsc

_N = 10000
_E = 160000
_NIN = 128
_NOUT = 128
_CB = 32
_K = 27

_NC = 2
_NS = 16
_L = 16

_EPAD = 163840
_EPT = _EPAD // (_NC * _NS)
_CHUNK = 128
_NCH = _EPT // _CHUNK
_NBUF = 4
_NACC = 10240
_JUNK = 10200
_RPT = _NACC // _NS
_ZR = 64
_NT = 7


def _tc1_body(x_ref, w1_ref, b1_ref, w2_ref, ht_ref):
    h = jnp.dot(x_ref[...], w1_ref[...], preferred_element_type=jnp.float32)
    h = jnp.maximum(h + b1_ref[...], 0.0)
    for t in range(_NT):
        ht_ref[t] = jnp.dot(h, w2_ref[:, pl.ds(t * 128, 128)],
                            preferred_element_type=jnp.float32)


def _tc2_body(p_ref, x_ref, b2_ref, w3_ref, b3_ref, o_ref):
    h2 = jnp.maximum(p_ref[0] + p_ref[1] + b2_ref[...], 0.0)
    h3 = jnp.dot(h2, w3_ref[...], preferred_element_type=jnp.float32)
    h3 = h3 + b3_ref[...]
    o_ref[...] = jnp.maximum((h3 + x_ref[...]) * 0.5, 0.0)


def _sc_body(ht_hbm, src_hbm, kidx_hbm, dst_hbm, out_hbm,
             srcbuf, kidxbuf, gidx, dstbuf, rows, zbuf, acc,
             sg0, sg1, sg2, sg3, lsem):
    sems = (sg0, sg1, sg2, sg3)
    c = lax.axis_index("c")
    s = lax.axis_index("s")
    wid = c * _NS + s
    ebase = wid * _EPT

    pltpu.async_copy(src_hbm.at[pl.ds(ebase, _EPT)], srcbuf, lsem)
    pltpu.async_copy(kidx_hbm.at[pl.ds(ebase, _EPT)], kidxbuf, lsem)
    pltpu.async_copy(dst_hbm.at[pl.ds(wid * _NCH, _NCH)], dstbuf, lsem)

    z16 = jnp.zeros((_L,), jnp.float32)

    def zero_zbuf(r, carry):
        zbuf[r, pl.ds(0, _L)] = z16
        zbuf[r, pl.ds(_L, _L)] = z16
        return carry
    lax.fori_loop(0, _ZR, zero_zbuf, 0)

    pltpu.make_async_copy(src_hbm.at[pl.ds(0, _EPT)], srcbuf, lsem).wait()
    pltpu.make_async_copy(src_hbm.at[pl.ds(0, _EPT)], kidxbuf, lsem).wait()
    pltpu.make_async_copy(dst_hbm.at[pl.ds(0, _NCH)], dstbuf, lsem).wait()

    for t in range(_RPT // _ZR):
        pltpu.async_copy(zbuf, acc.at[pl.ds(s * _RPT + t * _ZR, _ZR)], lsem)

    def build_idx(i, carry):
        o = i * _L
        kv = kidxbuf[pl.ds(o, _L)]
        sv = srcbuf[pl.ds(o, _L)]
        gidx[pl.ds(o, _L)] = (
            lax.shift_right_logical(kv, 2) * (4 * _N)
            + sv * 4 + lax.bitwise_and(kv, 3))
        return carry
    lax.fori_loop(0, _EPT // _L, build_idx, 0)

    def gather(j, b):
        pltpu.async_copy(ht_hbm.at[gidx.at[pl.ds(j * _CHUNK, _CHUNK)]],
                         rows.at[b], sems[b])

    def wait_gather(b):
        pltpu.make_async_copy(ht_hbm.at[pl.ds(0, _CHUNK)], rows.at[b],
                              sems[b]).wait()

    for b in range(_NBUF):
        gather(b, b)

    for t in range(_RPT // _ZR):
        pltpu.make_async_copy(ht_hbm.at[pl.ds(0, _ZR)], zbuf, lsem).wait()
    plsc.subcore_barrier()

    def chunk_round(jj, carry):
        j = jj * _NBUF
        for b in range(_NBUF):
            wait_gather(b)
            pltpu.sync_copy(rows.at[b], acc.at[dstbuf.at[j + b]], add=True)

            @pl.when(j + b + _NBUF < _NCH)
            def _():
                gather(j + b + _NBUF, b)
        return carry
    lax.fori_loop(0, _NCH // _NBUF, chunk_round, 0)
    plsc.subcore_barrier()

    pltpu.sync_copy(acc.at[pl.ds(s * _RPT, _RPT)],
                    out_hbm.at[pl.ds(c * _NACC + s * _RPT, _RPT)])


@jax.jit
def kernel(x, W1, b1, W2, b2, W3, b3, edge_src, edge_dst, edge_kidx):
    npad = _EPAD - _E
    src_p = jnp.concatenate([edge_src, jnp.zeros((npad,), jnp.int32)])
    kidx_p = jnp.concatenate([edge_kidx, jnp.zeros((npad,), jnp.int32)])
    dst_p = jnp.concatenate(
        [edge_dst, jnp.full((npad,), _JUNK, jnp.int32)]).reshape(-1, _CHUNK)

    bn1 = 512
    grid1 = pl.cdiv(_N, bn1)
    w2all = jnp.transpose(W2, (1, 0, 2)).reshape(_CB, _K * _CB)
    w2pad = jnp.concatenate(
        [w2all, jnp.zeros((_CB, _NT * 128 - _K * _CB), jnp.float32)], axis=1)
    ht = pl.pallas_call(
        _tc1_body,
        grid=(grid1,),
        in_specs=[
            pl.BlockSpec((bn1, _NIN), lambda i: (i, 0)),
            pl.BlockSpec((_NIN, _CB), lambda i: (0, 0)),
            pl.BlockSpec((1, _CB), lambda i: (0, 0)),
            pl.BlockSpec((_CB, _NT * 128), lambda i: (0, 0)),
        ],
        out_specs=pl.BlockSpec((_NT, bn1, 128), lambda i: (0, i, 0)),
        out_shape=jax.ShapeDtypeStruct((_NT, _N, 128), jnp.float32),
    )(x, W1, b1.reshape(1, _CB), w2pad)
    ht_rows = ht.reshape(_NT * _N * 4, _CB)

    mesh = plsc.VectorSubcoreMesh(core_axis_name="c", subcore_axis_name="s")
    partial = pl.kernel(
        _sc_body,
        out_type=jax.ShapeDtypeStruct((_NC * _NACC, _CB), jnp.float32),
        mesh=mesh,
        compiler_params=pltpu.CompilerParams(use_tc_tiling_on_sc=False),
        scratch_types=[
            pltpu.VMEM((_EPT,), jnp.int32),
            pltpu.VMEM((_EPT,), jnp.int32),
            pltpu.VMEM((_EPT,), jnp.int32),
            pltpu.VMEM((_NCH, _CHUNK), jnp.int32),
            pltpu.VMEM((_NBUF, _CHUNK, _CB), jnp.float32),
            pltpu.VMEM((_ZR, _CB), jnp.float32),
            pltpu.VMEM_SHARED((_NACC, _CB), jnp.float32),
            pltpu.SemaphoreType.DMA,
            pltpu.SemaphoreType.DMA,
            pltpu.SemaphoreType.DMA,
            pltpu.SemaphoreType.DMA,
            pltpu.SemaphoreType.DMA,
        ],
    )(ht_rows, src_p, kidx_p, dst_p)
    psum = partial.reshape(_NC, _NACC, _CB)

    bn2 = 512
    grid2 = pl.cdiv(_N, bn2)
    out = pl.pallas_call(
        _tc2_body,
        grid=(grid2,),
        in_specs=[
            pl.BlockSpec((_NC, bn2, _CB), lambda i: (0, i, 0)),
            pl.BlockSpec((bn2, _NIN), lambda i: (i, 0)),
            pl.BlockSpec((1, _CB), lambda i: (0, 0)),
            pl.BlockSpec((_CB, _NOUT), lambda i: (0, 0)),
            pl.BlockSpec((1, _NOUT), lambda i: (0, 0)),
        ],
        out_specs=pl.BlockSpec((bn2, _NOUT), lambda i: (i, 0)),
        out_shape=jax.ShapeDtypeStruct((_N, _NOUT), jnp.float32),
    )(psum, x, b2.reshape(1, _CB), W3, b3.reshape(1, _NOUT))
    return out


# speedup vs baseline: 9.3049x; 1.0302x over previous
import functools

import jax
import jax.numpy as jnp
from jax import lax
from jax.experimental import pallas as pl
from jax.experimental.pallas import tpu as pltpu
from jax.experimental.pallas import tpu_sc as plsc

_N = 10000
_E = 160000
_NIN = 128
_NOUT = 128
_CB = 32
_K = 27

_NC = 2
_NS = 16
_L = 16

_EPAD = 163840
_EALLOC = 167936
_CHUNK = 128
_NCH0 = 56
_NCH1 = 24
_EPT0 = _NCH0 * _CHUNK
_EPT1 = _NCH1 * _CHUNK
_NBUF = 4
_NACC = 10240
_JUNK = 10200
_RPT = _NACC // _NS
_ZR = 64
_NT = 7


def _tc1_body(x_ref, w1_ref, b1_ref, w2_ref, ht_ref):
    h = jnp.dot(x_ref[...], w1_ref[...], preferred_element_type=jnp.float32)
    h = jnp.maximum(h + b1_ref[...], 0.0)
    for t in range(_NT):
        ht_ref[t] = jnp.dot(h, w2_ref[:, pl.ds(t * 128, 128)],
                            preferred_element_type=jnp.float32)


def _tc2_body(p_ref, x_ref, b2_ref, w3_ref, b3_ref, o_ref):
    h2 = jnp.maximum(p_ref[0] + p_ref[1] + b2_ref[...], 0.0)
    h3 = jnp.dot(h2, w3_ref[...], preferred_element_type=jnp.float32)
    o_ref[...] = jnp.maximum((h3 + b3_ref[...] + x_ref[...]) * 0.5, 0.0)


def _sc_body(ht_hbm, src_hbm, kidx_hbm, dst_hbm, out_hbm,
             srcbuf, kidxbuf, gidx, dstbuf, rows, zbuf, acc,
             sg0, sg1, sg2, sg3, lsem):
    sems = (sg0, sg1, sg2, sg3)
    c = lax.axis_index("c")
    s = lax.axis_index("s")
    nch = jnp.where(c == 0, _NCH0, _NCH1)
    ebase = jnp.where(c == 0, s * _EPT0, _NS * _EPT0 + s * _EPT1)

    pltpu.async_copy(src_hbm.at[pl.ds(ebase, _EPT0)], srcbuf, lsem)
    pltpu.async_copy(kidx_hbm.at[pl.ds(ebase, _EPT0)], kidxbuf, lsem)
    pltpu.async_copy(dst_hbm.at[pl.ds(ebase // _CHUNK, _NCH0)], dstbuf, lsem)

    z16 = jnp.zeros((_L,), jnp.float32)

    def zero_zbuf(r, carry):
        zbuf[r, pl.ds(0, _L)] = z16
        zbuf[r, pl.ds(_L, _L)] = z16
        return carry
    lax.fori_loop(0, _ZR, zero_zbuf, 0)

    pltpu.make_async_copy(src_hbm.at[pl.ds(0, _EPT0)], srcbuf, lsem).wait()
    pltpu.make_async_copy(src_hbm.at[pl.ds(0, _EPT0)], kidxbuf, lsem).wait()
    pltpu.make_async_copy(dst_hbm.at[pl.ds(0, _NCH0)], dstbuf, lsem).wait()

    for t in range(_RPT // _ZR):
        pltpu.async_copy(zbuf, acc.at[pl.ds(s * _RPT + t * _ZR, _ZR)], lsem)

    def build_idx(i, carry):
        o = i * _L
        kv = kidxbuf[pl.ds(o, _L)]
        sv = srcbuf[pl.ds(o, _L)]
        gidx[pl.ds(o, _L)] = (
            lax.shift_right_logical(kv, 2) * (4 * _N)
            + sv * 4 + lax.bitwise_and(kv, 3))
        return carry
    lax.fori_loop(0, _EPT0 // _L, build_idx, 0)

    def gather(j, b):
        pltpu.async_copy(ht_hbm.at[gidx.at[pl.ds(j * _CHUNK, _CHUNK)]],
                         rows.at[b], sems[b])

    def wait_gather(b):
        pltpu.make_async_copy(ht_hbm.at[pl.ds(0, _CHUNK)], rows.at[b],
                              sems[b]).wait()

    for b in range(_NBUF):
        gather(b, b)

    for t in range(_RPT // _ZR):
        pltpu.make_async_copy(ht_hbm.at[pl.ds(0, _ZR)], zbuf, lsem).wait()
    plsc.subcore_barrier()

    def chunk_round(jj, carry):
        j = jj * _NBUF
        for b in range(_NBUF):
            wait_gather(b)
            pltpu.sync_copy(rows.at[b], acc.at[dstbuf.at[j + b]], add=True)

            @pl.when(j + b + _NBUF < nch)
            def _():
                gather(j + b + _NBUF, b)
        return carry
    lax.fori_loop(0, nch // _NBUF, chunk_round, 0)
    plsc.subcore_barrier()

    pltpu.sync_copy(acc.at[pl.ds(s * _RPT, _RPT)],
                    out_hbm.at[pl.ds(c * _NACC + s * _RPT, _RPT)])


@jax.jit
def kernel(x, W1, b1, W2, b2, W3, b3, edge_src, edge_dst, edge_kidx):
    npad = _EALLOC - _E
    src_p = jnp.concatenate([edge_src, jnp.zeros((npad,), jnp.int32)])
    kidx_p = jnp.concatenate([edge_kidx, jnp.zeros((npad,), jnp.int32)])
    dst_p = jnp.concatenate(
        [edge_dst, jnp.full((npad,), _JUNK, jnp.int32)]).reshape(-1, _CHUNK)

    bn1 = 512
    grid1 = pl.cdiv(_N, bn1)
    w2all = jnp.transpose(W2, (1, 0, 2)).reshape(_CB, _K * _CB)
    w2pad = jnp.concatenate(
        [w2all, jnp.zeros((_CB, _NT * 128 - _K * _CB), jnp.float32)], axis=1)
    ht = pl.pallas_call(
        _tc1_body,
        grid=(grid1,),
        in_specs=[
            pl.BlockSpec((bn1, _NIN), lambda i: (i, 0)),
            pl.BlockSpec((_NIN, _CB), lambda i: (0, 0)),
            pl.BlockSpec((1, _CB), lambda i: (0, 0)),
            pl.BlockSpec((_CB, _NT * 128), lambda i: (0, 0)),
        ],
        out_specs=pl.BlockSpec((_NT, bn1, 128), lambda i: (0, i, 0)),
        out_shape=jax.ShapeDtypeStruct((_NT, _N, 128), jnp.float32),
    )(x, W1, b1.reshape(1, _CB), w2pad)
    ht_rows = ht.reshape(_NT * _N * 4, _CB)

    mesh = plsc.VectorSubcoreMesh(core_axis_name="c", subcore_axis_name="s")
    partial = pl.kernel(
        _sc_body,
        out_type=jax.ShapeDtypeStruct((_NC * _NACC, _CB), jnp.float32),
        mesh=mesh,
        compiler_params=pltpu.CompilerParams(use_tc_tiling_on_sc=False),
        scratch_types=[
            pltpu.VMEM((_EPT0,), jnp.int32),
            pltpu.VMEM((_EPT0,), jnp.int32),
            pltpu.VMEM((_EPT0,), jnp.int32),
            pltpu.VMEM((_NCH0, _CHUNK), jnp.int32),
            pltpu.VMEM((_NBUF, _CHUNK, _CB), jnp.float32),
            pltpu.VMEM((_ZR, _CB), jnp.float32),
            pltpu.VMEM_SHARED((_NACC, _CB), jnp.float32),
            pltpu.SemaphoreType.DMA,
            pltpu.SemaphoreType.DMA,
            pltpu.SemaphoreType.DMA,
            pltpu.SemaphoreType.DMA,
            pltpu.SemaphoreType.DMA,
        ],
    )(ht_rows, src_p, kidx_p, dst_p)
    psum = partial.reshape(_NC, _NACC // 4, 4 * _CB)
    x4 = x.reshape(_N // 4, 4 * _NIN)
    w3big = jnp.kron(jnp.eye(4, dtype=jnp.float32), W3)
    b2t = jnp.tile(b2, 4).reshape(1, 4 * _CB)
    b3t = jnp.tile(b3, 4).reshape(1, 4 * _NOUT)

    bn2 = 128
    grid2 = pl.cdiv(_N // 4, bn2)
    out4 = pl.pallas_call(
        _tc2_body,
        grid=(grid2,),
        in_specs=[
            pl.BlockSpec((_NC, bn2, 4 * _CB), lambda i: (0, i, 0)),
            pl.BlockSpec((bn2, 4 * _NIN), lambda i: (i, 0)),
            pl.BlockSpec((1, 4 * _CB), lambda i: (0, 0)),
            pl.BlockSpec((4 * _CB, 4 * _NOUT), lambda i: (0, 0)),
            pl.BlockSpec((1, 4 * _NOUT), lambda i: (0, 0)),
        ],
        out_specs=pl.BlockSpec((bn2, 4 * _NOUT), lambda i: (i, 0)),
        out_shape=jax.ShapeDtypeStruct((_N // 4, 4 * _NOUT), jnp.float32),
    )(psum, x4, b2t, w3big, b3t)
    return out4.reshape(_N, _NOUT)
